# initial kernel scaffold (unmeasured)
import jax
import jax.numpy as jnp
from jax import lax
from jax.experimental import pallas as pl
from jax.experimental.pallas import tpu as pltpu

N_DEV = 8
B = 2
SQ = 512
SKV = 4096
SKV_LOC = SKV // N_DEV
HQ_LOC = 8
DH = 64
DM = 768
SQ_RS = SQ // N_DEV

_MESH = pl.DeviceIdType.MESH


def _compiler_params():
    cp = getattr(pltpu, "CompilerParams", None) or getattr(
        pltpu, "TPUCompilerParams"
    )
    return cp(collective_id=0)


def kernel(x, Wq, K_ext, V_ext, Wo):
    xb = x.astype(jnp.bfloat16)
    wqb = Wq.astype(jnp.bfloat16)
    kb = K_ext.astype(jnp.bfloat16)
    vb = V_ext.astype(jnp.bfloat16)
    wob = Wo.astype(jnp.bfloat16)

    def body(
        x_ref, wq_ref, k_ref, v_ref, wo_ref, out_ref,
        kall, vall, q_ref, ctx_ref, part_ref, rs_ref,
        k_send_sems, k_recv_sems, v_send_sems, v_recv_sems,
        rs_send_sems, rs_recv_sems, ag_send_sems, ag_recv_sems,
        local_sems,
    ):
        me = lax.axis_index("i")

        def peer(off):
            return lax.rem(me + off, N_DEV)

        barrier = pltpu.get_barrier_semaphore()
        for off in range(1, N_DEV):
            pl.semaphore_signal(
                barrier, inc=1, device_id=(peer(off),), device_id_type=_MESH
            )
        pl.semaphore_wait(barrier, N_DEV - 1)

        def a2a_kv(t):
            k_rdma = pltpu.make_async_remote_copy(
                src_ref=k_ref.at[:, :, pl.ds(t * HQ_LOC, HQ_LOC), :],
                dst_ref=kall.at[:, pl.ds(me * SKV_LOC, SKV_LOC), :, :],
                send_sem=k_send_sems.at[t],
                recv_sem=k_recv_sems.at[me],
                device_id=(t,),
                device_id_type=_MESH,
            )
            v_rdma = pltpu.make_async_remote_copy(
                src_ref=v_ref.at[:, :, pl.ds(t * HQ_LOC, HQ_LOC), :],
                dst_ref=vall.at[:, pl.ds(me * SKV_LOC, SKV_LOC), :, :],
                send_sem=v_send_sems.at[t],
                recv_sem=v_recv_sems.at[me],
                device_id=(t,),
                device_id_type=_MESH,
            )
            return k_rdma, v_rdma

        for off in range(1, N_DEV):
            kr, vr = a2a_kv(peer(off))
            kr.start()
            vr.start()

        kloc = pltpu.make_async_copy(
            k_ref.at[:, :, pl.ds(me * HQ_LOC, HQ_LOC), :],
            kall.at[:, pl.ds(me * SKV_LOC, SKV_LOC), :, :],
            local_sems.at[0],
        )
        vloc = pltpu.make_async_copy(
            v_ref.at[:, :, pl.ds(me * HQ_LOC, HQ_LOC), :],
            vall.at[:, pl.ds(me * SKV_LOC, SKV_LOC), :, :],
            local_sems.at[1],
        )
        kloc.start()
        vloc.start()

        for b in range(B):
            q_ref[b] = lax.dot_general(
                x_ref[b], wq_ref[...],
                (((1,), (0,)), ((), ())),
                preferred_element_type=jnp.float32,
            ).astype(jnp.bfloat16)

        kloc.wait()
        vloc.wait()

        for off in range(1, N_DEV):
            s = peer(off)
            for sems, dst in (
                (k_recv_sems, kall),
                (v_recv_sems, vall),
            ):
                pltpu.make_async_remote_copy(
                    src_ref=k_ref.at[:, :, pl.ds(s * HQ_LOC, HQ_LOC), :],
                    dst_ref=dst.at[:, pl.ds(s * SKV_LOC, SKV_LOC), :, :],
                    send_sem=k_send_sems.at[s],
                    recv_sem=sems.at[s],
                    device_id=(s,),
                    device_id_type=_MESH,
                ).wait_recv()

        qi = lax.broadcasted_iota(jnp.int32, (SQ, SKV), 0)
        ki = lax.broadcasted_iota(jnp.int32, (SQ, SKV), 1)
        mask = (jnp.abs(qi - ki) <= 128) | (ki < 32) | (qi < 32)

        for b in range(B):
            for h in range(HQ_LOC):
                q = q_ref[b, :, h * DH:(h + 1) * DH]
                k = kall[b, :, h, :]
                s = lax.dot_general(
                    q, k, (((1,), (1,)), ((), ())),
                    preferred_element_type=jnp.float32,
                ) * 0.125
                s = jnp.where(mask, s, -1e9)
                m = jnp.max(s, axis=-1, keepdims=True)
                e = jnp.exp(s - m)
                l = jnp.sum(e, axis=-1, keepdims=True)
                p = (e / l).astype(jnp.bfloat16)
                v = vall[b, :, h, :]
                ctx_ref[b, :, h * DH:(h + 1) * DH] = lax.dot_general(
                    p, v, (((1,), (0,)), ((), ())),
                    preferred_element_type=jnp.float32,
                )

        for b in range(B):
            part_ref[b] = lax.dot_general(
                ctx_ref[b].astype(jnp.bfloat16), wo_ref[...],
                (((1,), (0,)), ((), ())),
                preferred_element_type=jnp.float32,
            )

        for off in range(1, N_DEV):
            t = peer(off)
            pltpu.make_async_remote_copy(
                src_ref=part_ref.at[:, pl.ds(t * SQ_RS, SQ_RS), :],
                dst_ref=rs_ref.at[me],
                send_sem=rs_send_sems.at[t],
                recv_sem=rs_recv_sems.at[me],
                device_id=(t,),
                device_id_type=_MESH,
            ).start()

        red = part_ref[:, pl.ds(me * SQ_RS, SQ_RS), :]
        for off in range(1, N_DEV):
            s = peer(off)
            pltpu.make_async_remote_copy(
                src_ref=part_ref.at[:, pl.ds(s * SQ_RS, SQ_RS), :],
                dst_ref=rs_ref.at[s],
                send_sem=rs_send_sems.at[s],
                recv_sem=rs_recv_sems.at[s],
                device_id=(s,),
                device_id_type=_MESH,
            ).wait_recv()
            red = red + rs_ref[s]
        out_ref[:, pl.ds(me * SQ_RS, SQ_RS), :] = red

        for off in range(1, N_DEV):
            t = peer(off)
            pltpu.make_async_remote_copy(
                src_ref=out_ref.at[:, pl.ds(me * SQ_RS, SQ_RS), :],
                dst_ref=out_ref.at[:, pl.ds(me * SQ_RS, SQ_RS), :],
                send_sem=ag_send_sems.at[t],
                recv_sem=ag_recv_sems.at[me],
                device_id=(t,),
                device_id_type=_MESH,
            ).start()
        for off in range(1, N_DEV):
            s = peer(off)
            pltpu.make_async_remote_copy(
                src_ref=out_ref.at[:, pl.ds(s * SQ_RS, SQ_RS), :],
                dst_ref=out_ref.at[:, pl.ds(s * SQ_RS, SQ_RS), :],
                send_sem=ag_send_sems.at[s],
                recv_sem=ag_recv_sems.at[s],
                device_id=(s,),
                device_id_type=_MESH,
            ).wait_recv()

        for off in range(1, N_DEV):
            t = peer(off)
            kr, vr = a2a_kv(t)
            kr.wait_send()
            vr.wait_send()
            pltpu.make_async_remote_copy(
                src_ref=part_ref.at[:, pl.ds(t * SQ_RS, SQ_RS), :],
                dst_ref=rs_ref.at[me],
                send_sem=rs_send_sems.at[t],
                recv_sem=rs_recv_sems.at[me],
                device_id=(t,),
                device_id_type=_MESH,
            ).wait_send()
            pltpu.make_async_remote_copy(
                src_ref=out_ref.at[:, pl.ds(me * SQ_RS, SQ_RS), :],
                dst_ref=out_ref.at[:, pl.ds(me * SQ_RS, SQ_RS), :],
                send_sem=ag_send_sems.at[t],
                recv_sem=ag_recv_sems.at[me],
                device_id=(t,),
                device_id_type=_MESH,
            ).wait_send()

    dma8 = pltpu.SemaphoreType.DMA((N_DEV,))
    return pl.pallas_call(
        body,
        out_shape=jax.ShapeDtypeStruct((B, SQ, DM), jnp.float32),
        in_specs=[
            pl.BlockSpec(memory_space=pltpu.VMEM),
            pl.BlockSpec(memory_space=pltpu.VMEM),
            pl.BlockSpec(memory_space=pltpu.ANY),
            pl.BlockSpec(memory_space=pltpu.ANY),
            pl.BlockSpec(memory_space=pltpu.VMEM),
        ],
        out_specs=pl.BlockSpec(memory_space=pltpu.VMEM),
        scratch_shapes=[
            pltpu.VMEM((B, SKV, HQ_LOC, DH), jnp.bfloat16),
            pltpu.VMEM((B, SKV, HQ_LOC, DH), jnp.bfloat16),
            pltpu.VMEM((B, SQ, HQ_LOC * DH), jnp.bfloat16),
            pltpu.VMEM((B, SQ, HQ_LOC * DH), jnp.float32),
            pltpu.VMEM((B, SQ, DM), jnp.float32),
            pltpu.VMEM((N_DEV, B, SQ_RS, DM), jnp.float32),
            dma8, dma8, dma8, dma8,
            dma8, dma8, dma8, dma8,
            pltpu.SemaphoreType.DMA((2,)),
        ],
        compiler_params=_compiler_params(),
    )(xb, wqb, kb, vb, wob)


# baseline (device time: 446835 ns/iter reference)
import os

import jax
import jax.numpy as jnp
from jax import lax
from jax.experimental import pallas as pl
from jax.experimental.pallas import tpu as pltpu

N_DEV = 8
B = 2
SQ = 512
SKV = 4096
SKV_LOC = SKV // N_DEV
HQ_LOC = 8
DH = 64
DM = 768
SQ_RS = SQ // N_DEV

_MESH = pl.DeviceIdType.MESH

_ANY = pl.ANY


def _compiler_params():
    cp = getattr(pltpu, "CompilerParams", None) or getattr(
        pltpu, "TPUCompilerParams"
    )
    return cp(collective_id=0, vmem_limit_bytes=56 * 1024 * 1024)


def kernel(x, Wq, K_ext, V_ext, Wo):
    xb = x.astype(jnp.bfloat16)
    wqb = Wq.astype(jnp.bfloat16)
    kb = (
        K_ext.astype(jnp.bfloat16)
        .reshape(B, SKV_LOC, N_DEV, HQ_LOC, DH)
        .transpose(2, 0, 3, 1, 4)
    )
    vb = (
        V_ext.astype(jnp.bfloat16)
        .reshape(B, SKV_LOC, N_DEV, HQ_LOC, DH)
        .transpose(2, 0, 3, 1, 4)
    )
    wob = Wo.astype(jnp.bfloat16).reshape(HQ_LOC, DH, DM)

    def body(
        x_ref, wq_ref, k_ref, v_ref, wo_ref, out_ref,
        kall, vall, q_ref, part_ref, rs_ref,
        k_send_sems, k_recv_sems, v_send_sems, v_recv_sems,
        rs_send_sems, rs_recv_sems, ag_send_sems, ag_recv_sems,
        local_sems,
    ):
        me = lax.axis_index("i")

        def peer(off):
            return lax.rem(me + off, N_DEV)

        barrier = pltpu.get_barrier_semaphore()
        for off in range(1, N_DEV):
            pl.semaphore_signal(
                barrier, inc=1, device_id=(peer(off),), device_id_type=_MESH
            )
        pl.semaphore_wait(barrier, N_DEV - 1)

        def a2a_kv(t):
            k_rdma = pltpu.make_async_remote_copy(
                src_ref=k_ref.at[t],
                dst_ref=kall.at[:, :, pl.ds(me * SKV_LOC, SKV_LOC), :],
                send_sem=k_send_sems.at[t],
                recv_sem=k_recv_sems.at[me],
                device_id=(t,),
                device_id_type=_MESH,
            )
            v_rdma = pltpu.make_async_remote_copy(
                src_ref=v_ref.at[t],
                dst_ref=vall.at[:, :, pl.ds(me * SKV_LOC, SKV_LOC), :],
                send_sem=v_send_sems.at[t],
                recv_sem=v_recv_sems.at[me],
                device_id=(t,),
                device_id_type=_MESH,
            )
            return k_rdma, v_rdma

        for off in range(1, N_DEV):
            kr, vr = a2a_kv(peer(off))
            kr.start()
            vr.start()

        kloc = pltpu.make_async_copy(
            k_ref.at[me],
            kall.at[:, :, pl.ds(me * SKV_LOC, SKV_LOC), :],
            local_sems.at[0],
        )
        vloc = pltpu.make_async_copy(
            v_ref.at[me],
            vall.at[:, :, pl.ds(me * SKV_LOC, SKV_LOC), :],
            local_sems.at[1],
        )
        kloc.start()
        vloc.start()

        for b in range(B):
            qb = lax.dot_general(
                x_ref[b], wq_ref[...],
                (((1,), (0,)), ((), ())),
                preferred_element_type=jnp.float32,
            ).astype(jnp.bfloat16)
            for h in range(HQ_LOC):
                q_ref[b, h] = qb[:, h * DH:(h + 1) * DH]

        kloc.wait()
        vloc.wait()

        for off in range(1, N_DEV):
            s = peer(off)
            for sems, dst in (
                (k_recv_sems, kall),
                (v_recv_sems, vall),
            ):
                pltpu.make_async_remote_copy(
                    src_ref=k_ref.at[s],
                    dst_ref=dst.at[:, :, pl.ds(s * SKV_LOC, SKV_LOC), :],
                    send_sem=k_send_sems.at[s],
                    recv_sem=sems.at[s],
                    device_id=(s,),
                    device_id_type=_MESH,
                ).wait_recv()

        qi = lax.broadcasted_iota(jnp.int32, (SQ, SKV), 0)
        ki = lax.broadcasted_iota(jnp.int32, (SQ, SKV), 1)
        mask = (jnp.abs(qi - ki) <= 128) | (ki < 32) | (qi < 32)

        part_ref[...] = jnp.zeros((B, SQ, DM), jnp.float32)

        def attn_step(idx, carry):
            b = idx // HQ_LOC
            h = lax.rem(idx, HQ_LOC)
            q = q_ref[pl.ds(b, 1), pl.ds(h, 1)].reshape((SQ, DH))
            k = kall[pl.ds(b, 1), pl.ds(h, 1)].reshape((SKV, DH))
            s = lax.dot_general(
                q, k, (((1,), (1,)), ((), ())),
                preferred_element_type=jnp.float32,
            ) * 0.125
            s = jnp.where(mask, s, -1e9)
            m = jnp.max(s, axis=-1, keepdims=True)
            e = jnp.exp(s - m)
            l = jnp.sum(e, axis=-1, keepdims=True)
            p = (e / l).astype(jnp.bfloat16)
            v = vall[pl.ds(b, 1), pl.ds(h, 1)].reshape((SKV, DH))
            c = lax.dot_general(
                p, v, (((1,), (0,)), ((), ())),
                preferred_element_type=jnp.float32,
            ).astype(jnp.bfloat16)
            w = wo_ref[pl.ds(h, 1)].reshape((DH, DM))
            delta = lax.dot_general(
                c, w, (((1,), (0,)), ((), ())),
                preferred_element_type=jnp.float32,
            )
            part_ref[pl.ds(b, 1)] = part_ref[pl.ds(b, 1)] + delta[None]
            return carry

        lax.fori_loop(0, B * HQ_LOC, attn_step, 0, unroll=False)

        for off in range(1, N_DEV):
            t = peer(off)
            pltpu.make_async_remote_copy(
                src_ref=part_ref.at[:, pl.ds(t * SQ_RS, SQ_RS), :],
                dst_ref=rs_ref.at[me],
                send_sem=rs_send_sems.at[t],
                recv_sem=rs_recv_sems.at[me],
                device_id=(t,),
                device_id_type=_MESH,
            ).start()

        red = part_ref[:, pl.ds(me * SQ_RS, SQ_RS), :]
        for off in range(1, N_DEV):
            s = peer(off)
            pltpu.make_async_remote_copy(
                src_ref=part_ref.at[:, pl.ds(s * SQ_RS, SQ_RS), :],
                dst_ref=rs_ref.at[s],
                send_sem=rs_send_sems.at[s],
                recv_sem=rs_recv_sems.at[s],
                device_id=(s,),
                device_id_type=_MESH,
            ).wait_recv()
            red = red + rs_ref[s]
        out_ref[:, pl.ds(me * SQ_RS, SQ_RS), :] = red

        for off in range(1, N_DEV):
            t = peer(off)
            pltpu.make_async_remote_copy(
                src_ref=out_ref.at[:, pl.ds(me * SQ_RS, SQ_RS), :],
                dst_ref=out_ref.at[:, pl.ds(me * SQ_RS, SQ_RS), :],
                send_sem=ag_send_sems.at[t],
                recv_sem=ag_recv_sems.at[me],
                device_id=(t,),
                device_id_type=_MESH,
            ).start()
        for off in range(1, N_DEV):
            s = peer(off)
            pltpu.make_async_remote_copy(
                src_ref=out_ref.at[:, pl.ds(s * SQ_RS, SQ_RS), :],
                dst_ref=out_ref.at[:, pl.ds(s * SQ_RS, SQ_RS), :],
                send_sem=ag_send_sems.at[s],
                recv_sem=ag_recv_sems.at[s],
                device_id=(s,),
                device_id_type=_MESH,
            ).wait_recv()

        for off in range(1, N_DEV):
            t = peer(off)
            kr, vr = a2a_kv(t)
            kr.wait_send()
            vr.wait_send()
            pltpu.make_async_remote_copy(
                src_ref=part_ref.at[:, pl.ds(t * SQ_RS, SQ_RS), :],
                dst_ref=rs_ref.at[me],
                send_sem=rs_send_sems.at[t],
                recv_sem=rs_recv_sems.at[me],
                device_id=(t,),
                device_id_type=_MESH,
            ).wait_send()
            pltpu.make_async_remote_copy(
                src_ref=out_ref.at[:, pl.ds(me * SQ_RS, SQ_RS), :],
                dst_ref=out_ref.at[:, pl.ds(me * SQ_RS, SQ_RS), :],
                send_sem=ag_send_sems.at[t],
                recv_sem=ag_recv_sems.at[me],
                device_id=(t,),
                device_id_type=_MESH,
            ).wait_send()

    dma8 = pltpu.SemaphoreType.DMA((N_DEV,))
    return pl.pallas_call(
        body,
        out_shape=jax.ShapeDtypeStruct((B, SQ, DM), jnp.float32),
        in_specs=[
            pl.BlockSpec(memory_space=pltpu.VMEM),
            pl.BlockSpec(memory_space=pltpu.VMEM),
            pl.BlockSpec(memory_space=_ANY),
            pl.BlockSpec(memory_space=_ANY),
            pl.BlockSpec(memory_space=pltpu.VMEM),
        ],
        out_specs=pl.BlockSpec(memory_space=pltpu.VMEM),
        scratch_shapes=[
            pltpu.VMEM((B, HQ_LOC, SKV, DH), jnp.bfloat16),
            pltpu.VMEM((B, HQ_LOC, SKV, DH), jnp.bfloat16),
            pltpu.VMEM((B, HQ_LOC, SQ, DH), jnp.bfloat16),
            pltpu.VMEM((B, SQ, DM), jnp.float32),
            pltpu.VMEM((N_DEV, B, SQ_RS, DM), jnp.float32),
            dma8, dma8, dma8, dma8,
            dma8, dma8, dma8, dma8,
            pltpu.SemaphoreType.DMA((2,)),
        ],
        compiler_params=_compiler_params(),
        interpret=(
            pltpu.InterpretParams()
            if os.environ.get("KERNEL_INTERPRET") == "1"
            else False
        ),
    )(xb, wqb, kb, vb, wob)


# device time: 408099 ns/iter; 1.0949x vs baseline; 1.0949x over previous
import os

import jax
import jax.numpy as jnp
from jax import lax
from jax.experimental import pallas as pl
from jax.experimental.pallas import tpu as pltpu

N_DEV = 8
B = 2
SQ = 512
SKV = 4096
SKV_LOC = SKV // N_DEV
HQ_LOC = 8
DH = 64
DM = 768
SQ_RS = SQ // N_DEV

_MESH = pl.DeviceIdType.MESH

_ANY = pl.ANY


def _compiler_params():
    cp = getattr(pltpu, "CompilerParams", None) or getattr(
        pltpu, "TPUCompilerParams"
    )
    return cp(collective_id=0, vmem_limit_bytes=56 * 1024 * 1024)


def kernel(x, Wq, K_ext, V_ext, Wo):
    xb = x.astype(jnp.bfloat16)
    wqb = Wq.astype(jnp.bfloat16)
    kb = (
        K_ext.astype(jnp.bfloat16)
        .reshape(B, SKV_LOC, N_DEV, HQ_LOC, DH)
        .transpose(2, 0, 3, 1, 4)
    )
    vb = (
        V_ext.astype(jnp.bfloat16)
        .reshape(B, SKV_LOC, N_DEV, HQ_LOC, DH)
        .transpose(2, 0, 3, 1, 4)
    )
    wob = Wo.astype(jnp.bfloat16).reshape(HQ_LOC, DH, DM)

    def body(
        x_ref, wq_ref, k_ref, v_ref, wo_ref, out_ref,
        kall, vall, q_ref, part_ref, rs_ref,
        k_send_sems, k_recv_sems, v_send_sems, v_recv_sems,
        rs_send_sems, rs_recv_sems, ag_send_sems, ag_recv_sems,
        local_sems,
    ):
        me = lax.axis_index("i")

        def peer(off):
            return lax.rem(me + off, N_DEV)

        barrier = pltpu.get_barrier_semaphore()
        for off in range(1, N_DEV):
            pl.semaphore_signal(
                barrier, inc=1, device_id=(peer(off),), device_id_type=_MESH
            )
        pl.semaphore_wait(barrier, N_DEV - 1)

        def a2a_kv(t):
            k_rdma = pltpu.make_async_remote_copy(
                src_ref=k_ref.at[t],
                dst_ref=kall.at[:, :, pl.ds(me * SKV_LOC, SKV_LOC), :],
                send_sem=k_send_sems.at[t],
                recv_sem=k_recv_sems.at[me],
                device_id=(t,),
                device_id_type=_MESH,
            )
            v_rdma = pltpu.make_async_remote_copy(
                src_ref=v_ref.at[t],
                dst_ref=vall.at[:, :, pl.ds(me * SKV_LOC, SKV_LOC), :],
                send_sem=v_send_sems.at[t],
                recv_sem=v_recv_sems.at[me],
                device_id=(t,),
                device_id_type=_MESH,
            )
            return k_rdma, v_rdma

        for off in range(1, N_DEV):
            kr, vr = a2a_kv(peer(off))
            kr.start()
            vr.start()

        kloc = pltpu.make_async_copy(
            k_ref.at[me],
            kall.at[:, :, pl.ds(me * SKV_LOC, SKV_LOC), :],
            local_sems.at[0],
        )
        vloc = pltpu.make_async_copy(
            v_ref.at[me],
            vall.at[:, :, pl.ds(me * SKV_LOC, SKV_LOC), :],
            local_sems.at[1],
        )
        kloc.start()
        vloc.start()

        for b in range(B):
            qb = (
                lax.dot_general(
                    x_ref[b], wq_ref[...],
                    (((1,), (0,)), ((), ())),
                    preferred_element_type=jnp.float32,
                )
                * 0.125
            ).astype(jnp.bfloat16)
            for h in range(HQ_LOC):
                q_ref[b, h] = qb[:, h * DH:(h + 1) * DH]

        kloc.wait()
        vloc.wait()

        for off in range(1, N_DEV):
            s = peer(off)
            for sems, dst in (
                (k_recv_sems, kall),
                (v_recv_sems, vall),
            ):
                pltpu.make_async_remote_copy(
                    src_ref=k_ref.at[s],
                    dst_ref=dst.at[:, :, pl.ds(s * SKV_LOC, SKV_LOC), :],
                    send_sem=k_send_sems.at[s],
                    recv_sem=sems.at[s],
                    device_id=(s,),
                    device_id_type=_MESH,
                ).wait_recv()

        KB = 1024
        GQ = 32
        qi1 = lax.broadcasted_iota(jnp.int32, (SQ, KB), 0)
        ki1 = lax.broadcasted_iota(jnp.int32, (SQ, KB), 1)
        mask1 = (jnp.abs(qi1 - ki1) <= 128) | (ki1 < 32) | (qi1 < 32)
        neg_pad = jnp.full((SQ - GQ, 1), -jnp.inf, jnp.float32)
        zero_pad = jnp.zeros((SQ - GQ, 1), jnp.float32)

        part_ref[...] = jnp.zeros((B, SQ, DM), jnp.float32)

        def attn_step(idx, carry):
            b = idx // HQ_LOC
            h = lax.rem(idx, HQ_LOC)
            q = q_ref[pl.ds(b, 1), pl.ds(h, 1)].reshape((SQ, DH))
            k = kall[pl.ds(b, 1), pl.ds(h, 1)].reshape((SKV, DH))
            v = vall[pl.ds(b, 1), pl.ds(h, 1)].reshape((SKV, DH))
            k1, k2 = k[:KB], k[KB:]
            v1, v2 = v[:KB], v[KB:]
            s1 = lax.dot_general(
                q, k1, (((1,), (1,)), ((), ())),
                preferred_element_type=jnp.float32,
            )
            s1 = jnp.where(mask1, s1, -jnp.inf)
            s2 = lax.dot_general(
                q[:GQ], k2, (((1,), (1,)), ((), ())),
                preferred_element_type=jnp.float32,
            )
            m1 = jnp.max(s1, axis=-1, keepdims=True)
            m2 = jnp.max(s2, axis=-1, keepdims=True)
            m = jnp.maximum(m1, jnp.concatenate([m2, neg_pad]))
            e1 = jnp.exp(s1 - m)
            e2 = jnp.exp(s2 - m[:GQ])
            l = jnp.sum(e1, axis=-1, keepdims=True) + jnp.concatenate(
                [jnp.sum(e2, axis=-1, keepdims=True), zero_pad]
            )
            r = 1.0 / l
            p1 = (e1 * r).astype(jnp.bfloat16)
            p2 = (e2 * r[:GQ]).astype(jnp.bfloat16)
            c = lax.dot_general(
                p1, v1, (((1,), (0,)), ((), ())),
                preferred_element_type=jnp.float32,
            )
            c2 = lax.dot_general(
                p2, v2, (((1,), (0,)), ((), ())),
                preferred_element_type=jnp.float32,
            )
            c = (
                c + jnp.concatenate([c2, jnp.zeros((SQ - GQ, DH), jnp.float32)])
            ).astype(jnp.bfloat16)
            w = wo_ref[pl.ds(h, 1)].reshape((DH, DM))
            delta = lax.dot_general(
                c, w, (((1,), (0,)), ((), ())),
                preferred_element_type=jnp.float32,
            )
            part_ref[pl.ds(b, 1)] = part_ref[pl.ds(b, 1)] + delta[None]
            return carry

        lax.fori_loop(0, B * HQ_LOC, attn_step, 0, unroll=False)

        for off in range(1, N_DEV):
            t = peer(off)
            pltpu.make_async_remote_copy(
                src_ref=part_ref.at[:, pl.ds(t * SQ_RS, SQ_RS), :],
                dst_ref=rs_ref.at[me],
                send_sem=rs_send_sems.at[t],
                recv_sem=rs_recv_sems.at[me],
                device_id=(t,),
                device_id_type=_MESH,
            ).start()

        red = part_ref[:, pl.ds(me * SQ_RS, SQ_RS), :]
        for off in range(1, N_DEV):
            s = peer(off)
            pltpu.make_async_remote_copy(
                src_ref=part_ref.at[:, pl.ds(s * SQ_RS, SQ_RS), :],
                dst_ref=rs_ref.at[s],
                send_sem=rs_send_sems.at[s],
                recv_sem=rs_recv_sems.at[s],
                device_id=(s,),
                device_id_type=_MESH,
            ).wait_recv()
            red = red + rs_ref[s]
        out_ref[:, pl.ds(me * SQ_RS, SQ_RS), :] = red

        for off in range(1, N_DEV):
            t = peer(off)
            pltpu.make_async_remote_copy(
                src_ref=out_ref.at[:, pl.ds(me * SQ_RS, SQ_RS), :],
                dst_ref=out_ref.at[:, pl.ds(me * SQ_RS, SQ_RS), :],
                send_sem=ag_send_sems.at[t],
                recv_sem=ag_recv_sems.at[me],
                device_id=(t,),
                device_id_type=_MESH,
            ).start()
        for off in range(1, N_DEV):
            s = peer(off)
            pltpu.make_async_remote_copy(
                src_ref=out_ref.at[:, pl.ds(s * SQ_RS, SQ_RS), :],
                dst_ref=out_ref.at[:, pl.ds(s * SQ_RS, SQ_RS), :],
                send_sem=ag_send_sems.at[s],
                recv_sem=ag_recv_sems.at[s],
                device_id=(s,),
                device_id_type=_MESH,
            ).wait_recv()

        for off in range(1, N_DEV):
            t = peer(off)
            kr, vr = a2a_kv(t)
            kr.wait_send()
            vr.wait_send()
            pltpu.make_async_remote_copy(
                src_ref=part_ref.at[:, pl.ds(t * SQ_RS, SQ_RS), :],
                dst_ref=rs_ref.at[me],
                send_sem=rs_send_sems.at[t],
                recv_sem=rs_recv_sems.at[me],
                device_id=(t,),
                device_id_type=_MESH,
            ).wait_send()
            pltpu.make_async_remote_copy(
                src_ref=out_ref.at[:, pl.ds(me * SQ_RS, SQ_RS), :],
                dst_ref=out_ref.at[:, pl.ds(me * SQ_RS, SQ_RS), :],
                send_sem=ag_send_sems.at[t],
                recv_sem=ag_recv_sems.at[me],
                device_id=(t,),
                device_id_type=_MESH,
            ).wait_send()

    dma8 = pltpu.SemaphoreType.DMA((N_DEV,))
    return pl.pallas_call(
        body,
        out_shape=jax.ShapeDtypeStruct((B, SQ, DM), jnp.float32),
        in_specs=[
            pl.BlockSpec(memory_space=pltpu.VMEM),
            pl.BlockSpec(memory_space=pltpu.VMEM),
            pl.BlockSpec(memory_space=_ANY),
            pl.BlockSpec(memory_space=_ANY),
            pl.BlockSpec(memory_space=pltpu.VMEM),
        ],
        out_specs=pl.BlockSpec(memory_space=pltpu.VMEM),
        scratch_shapes=[
            pltpu.VMEM((B, HQ_LOC, SKV, DH), jnp.bfloat16),
            pltpu.VMEM((B, HQ_LOC, SKV, DH), jnp.bfloat16),
            pltpu.VMEM((B, HQ_LOC, SQ, DH), jnp.bfloat16),
            pltpu.VMEM((B, SQ, DM), jnp.float32),
            pltpu.VMEM((N_DEV, B, SQ_RS, DM), jnp.float32),
            dma8, dma8, dma8, dma8,
            dma8, dma8, dma8, dma8,
            pltpu.SemaphoreType.DMA((2,)),
        ],
        compiler_params=_compiler_params(),
        interpret=(
            pltpu.InterpretParams()
            if os.environ.get("KERNEL_INTERPRET") == "1"
            else False
        ),
    )(xb, wqb, kb, vb, wob)


# device time: 366250 ns/iter; 1.2200x vs baseline; 1.1143x over previous
import os

import jax
import jax.numpy as jnp
from jax import lax
from jax.experimental import pallas as pl
from jax.experimental.pallas import tpu as pltpu

N_DEV = 8
B = 2
SQ = 512
SKV = 4096
SKV_LOC = SKV // N_DEV
HQ_LOC = 8
DH = 64
DM = 768
SQ_RS = SQ // N_DEV

_MESH = pl.DeviceIdType.MESH

_ANY = pl.ANY


def _compiler_params():
    cp = getattr(pltpu, "CompilerParams", None) or getattr(
        pltpu, "TPUCompilerParams"
    )
    return cp(collective_id=0, vmem_limit_bytes=56 * 1024 * 1024)


def kernel(x, Wq, K_ext, V_ext, Wo):
    xb = x.astype(jnp.bfloat16)
    wqb = Wq.astype(jnp.bfloat16)
    kb = (
        K_ext.astype(jnp.bfloat16)
        .reshape(B, SKV_LOC, N_DEV, HQ_LOC, DH)
        .transpose(2, 0, 3, 1, 4)
    )
    vb = (
        V_ext.astype(jnp.bfloat16)
        .reshape(B, SKV_LOC, N_DEV, HQ_LOC, DH)
        .transpose(2, 0, 3, 1, 4)
    )
    wob = Wo.astype(jnp.bfloat16).reshape(HQ_LOC, DH, DM)

    def body(
        x_ref, wq_ref, k_ref, v_ref, wo_ref, out_ref,
        kall, vall, q_ref, part_ref, strip_ref, pbf_ref, rs_ref, ag_ref,
        k_send_sems, k_recv_sems, v_send_sems, v_recv_sems,
        rs_send_sems, rs_recv_sems, ag_send_sems, ag_recv_sems,
        local_sems,
    ):
        me = lax.axis_index("i")

        def peer(off):
            return lax.rem(me + off, N_DEV)

        barrier = pltpu.get_barrier_semaphore()
        for off in range(1, N_DEV):
            pl.semaphore_signal(
                barrier, inc=1, device_id=(peer(off),), device_id_type=_MESH
            )
        pl.semaphore_wait(barrier, N_DEV - 1)

        def a2a_kv(t):
            k_rdma = pltpu.make_async_remote_copy(
                src_ref=k_ref.at[t],
                dst_ref=kall.at[:, :, pl.ds(me * SKV_LOC, SKV_LOC), :],
                send_sem=k_send_sems.at[t],
                recv_sem=k_recv_sems.at[me],
                device_id=(t,),
                device_id_type=_MESH,
            )
            v_rdma = pltpu.make_async_remote_copy(
                src_ref=v_ref.at[t],
                dst_ref=vall.at[:, :, pl.ds(me * SKV_LOC, SKV_LOC), :],
                send_sem=v_send_sems.at[t],
                recv_sem=v_recv_sems.at[me],
                device_id=(t,),
                device_id_type=_MESH,
            )
            return k_rdma, v_rdma

        for off in range(1, N_DEV):
            kr, vr = a2a_kv(peer(off))
            kr.start()
            vr.start()

        kloc = pltpu.make_async_copy(
            k_ref.at[me],
            kall.at[:, :, pl.ds(me * SKV_LOC, SKV_LOC), :],
            local_sems.at[0],
        )
        vloc = pltpu.make_async_copy(
            v_ref.at[me],
            vall.at[:, :, pl.ds(me * SKV_LOC, SKV_LOC), :],
            local_sems.at[1],
        )
        kloc.start()
        vloc.start()

        for b in range(B):
            qb = (
                lax.dot_general(
                    x_ref[b], wq_ref[...],
                    (((1,), (0,)), ((), ())),
                    preferred_element_type=jnp.float32,
                )
                * 0.125
            ).astype(jnp.bfloat16)
            for h in range(HQ_LOC):
                q_ref[b, h] = qb[:, h * DH:(h + 1) * DH]

        kloc.wait()
        vloc.wait()

        def wait_kv(s):
            @pl.when(me != s)
            def _():
                for sems, dst in (
                    (k_recv_sems, kall),
                    (v_recv_sems, vall),
                ):
                    pltpu.make_async_remote_copy(
                        src_ref=k_ref.at[s],
                        dst_ref=dst.at[:, :, pl.ds(s * SKV_LOC, SKV_LOC), :],
                        send_sem=k_send_sems.at[s],
                        recv_sem=sems.at[s],
                        device_id=(s,),
                        device_id_type=_MESH,
                    ).wait_recv()

        KB = 1024
        GQ = 32
        qi1 = lax.broadcasted_iota(jnp.int32, (SQ, KB), 0)
        ki1 = lax.broadcasted_iota(jnp.int32, (SQ, KB), 1)
        mask1 = (jnp.abs(qi1 - ki1) <= 128) | (ki1 < 32) | (qi1 < 32)

        part_ref[...] = jnp.zeros((B, SQ, DM), jnp.float32)

        wait_kv(0)
        wait_kv(1)

        def band_step(idx, carry):
            b = idx // HQ_LOC
            h = lax.rem(idx, HQ_LOC)
            q = q_ref[pl.ds(b, 1), pl.ds(h, 1)].reshape((SQ, DH))
            k1 = kall[pl.ds(b, 1), pl.ds(h, 1), pl.ds(0, KB)].reshape((KB, DH))
            v1 = vall[pl.ds(b, 1), pl.ds(h, 1), pl.ds(0, KB)].reshape((KB, DH))
            s1 = lax.dot_general(
                q, k1, (((1,), (1,)), ((), ())),
                preferred_element_type=jnp.float32,
            )
            strip_ref[pl.ds(b, 1), pl.ds(h, 1)] = s1[:GQ][None, None]
            s1 = jnp.where(mask1, s1, -jnp.inf)
            m1 = jnp.max(s1, axis=-1, keepdims=True)
            e1 = jnp.exp(s1 - m1)
            r = 1.0 / jnp.sum(e1, axis=-1, keepdims=True)
            p1 = (e1 * r).astype(jnp.bfloat16)
            c = lax.dot_general(
                p1, v1, (((1,), (0,)), ((), ())),
                preferred_element_type=jnp.float32,
            )
            c = jnp.concatenate(
                [jnp.zeros((GQ, DH), jnp.float32), c[GQ:]]
            ).astype(jnp.bfloat16)
            w = wo_ref[pl.ds(h, 1)].reshape((DH, DM))
            delta = lax.dot_general(
                c, w, (((1,), (0,)), ((), ())),
                preferred_element_type=jnp.float32,
            )
            part_ref[pl.ds(b, 1)] = part_ref[pl.ds(b, 1)] + delta[None]
            return carry

        lax.fori_loop(0, B * HQ_LOC, band_step, 0, unroll=False)

        pbf_ref[...] = part_ref[...].astype(jnp.bfloat16)

        def rs_send(t):
            return pltpu.make_async_remote_copy(
                src_ref=pbf_ref.at[:, pl.ds(t * SQ_RS, SQ_RS), :],
                dst_ref=rs_ref.at[me],
                send_sem=rs_send_sems.at[t],
                recv_sem=rs_recv_sems.at[me],
                device_id=(t,),
                device_id_type=_MESH,
            )

        for off in range(1, N_DEV):
            t = peer(off)

            @pl.when(t != 0)
            def _():
                rs_send(t).start()

        for s in range(2, N_DEV):
            wait_kv(s)

        def strip_step(idx, carry):
            b = idx // HQ_LOC
            h = lax.rem(idx, HQ_LOC)
            q32 = q_ref[pl.ds(b, 1), pl.ds(h, 1), pl.ds(0, GQ)].reshape(
                (GQ, DH)
            )
            k2 = kall[pl.ds(b, 1), pl.ds(h, 1), pl.ds(KB, SKV - KB)].reshape(
                (SKV - KB, DH)
            )
            v2 = vall[pl.ds(b, 1), pl.ds(h, 1), pl.ds(KB, SKV - KB)].reshape(
                (SKV - KB, DH)
            )
            v1 = vall[pl.ds(b, 1), pl.ds(h, 1), pl.ds(0, KB)].reshape((KB, DH))
            s1 = strip_ref[pl.ds(b, 1), pl.ds(h, 1)].reshape((GQ, KB))
            s2 = lax.dot_general(
                q32, k2, (((1,), (1,)), ((), ())),
                preferred_element_type=jnp.float32,
            )
            m = jnp.maximum(
                jnp.max(s1, axis=-1, keepdims=True),
                jnp.max(s2, axis=-1, keepdims=True),
            )
            e1 = jnp.exp(s1 - m)
            e2 = jnp.exp(s2 - m)
            r = 1.0 / (
                jnp.sum(e1, axis=-1, keepdims=True)
                + jnp.sum(e2, axis=-1, keepdims=True)
            )
            p1 = (e1 * r).astype(jnp.bfloat16)
            p2 = (e2 * r).astype(jnp.bfloat16)
            c = (
                lax.dot_general(
                    p1, v1, (((1,), (0,)), ((), ())),
                    preferred_element_type=jnp.float32,
                )
                + lax.dot_general(
                    p2, v2, (((1,), (0,)), ((), ())),
                    preferred_element_type=jnp.float32,
                )
            ).astype(jnp.bfloat16)
            w = wo_ref[pl.ds(h, 1)].reshape((DH, DM))
            delta = lax.dot_general(
                c, w, (((1,), (0,)), ((), ())),
                preferred_element_type=jnp.float32,
            )
            part_ref[pl.ds(b, 1), pl.ds(0, GQ)] = (
                part_ref[pl.ds(b, 1), pl.ds(0, GQ)] + delta[None]
            )
            return carry

        lax.fori_loop(0, B * HQ_LOC, strip_step, 0, unroll=False)

        pbf_ref[:, pl.ds(0, SQ_RS), :] = part_ref[:, pl.ds(0, SQ_RS), :].astype(
            jnp.bfloat16
        )
        for off in range(1, N_DEV):
            t = peer(off)

            @pl.when(t == 0)
            def _():
                rs_send(t).start()

        red = part_ref[:, pl.ds(me * SQ_RS, SQ_RS), :]
        for off in range(1, N_DEV):
            s = peer(off)
            pltpu.make_async_remote_copy(
                src_ref=pbf_ref.at[:, pl.ds(s * SQ_RS, SQ_RS), :],
                dst_ref=rs_ref.at[s],
                send_sem=rs_send_sems.at[s],
                recv_sem=rs_recv_sems.at[s],
                device_id=(s,),
                device_id_type=_MESH,
            ).wait_recv()
            red = red + rs_ref[s].astype(jnp.float32)
        out_ref[:, pl.ds(me * SQ_RS, SQ_RS), :] = red
        ag_ref[pl.ds(me, 1)] = red.astype(jnp.bfloat16)[None]

        def ag_rdma(t):
            return pltpu.make_async_remote_copy(
                src_ref=ag_ref.at[me],
                dst_ref=ag_ref.at[me],
                send_sem=ag_send_sems.at[t],
                recv_sem=ag_recv_sems.at[me],
                device_id=(t,),
                device_id_type=_MESH,
            )

        for off in range(1, N_DEV):
            ag_rdma(peer(off)).start()
        for off in range(1, N_DEV):
            s = peer(off)
            pltpu.make_async_remote_copy(
                src_ref=ag_ref.at[s],
                dst_ref=ag_ref.at[s],
                send_sem=ag_send_sems.at[s],
                recv_sem=ag_recv_sems.at[s],
                device_id=(s,),
                device_id_type=_MESH,
            ).wait_recv()
            out_ref[:, pl.ds(s * SQ_RS, SQ_RS), :] = ag_ref[s].astype(
                jnp.float32
            )

        for off in range(1, N_DEV):
            t = peer(off)
            kr, vr = a2a_kv(t)
            kr.wait_send()
            vr.wait_send()
            rs_send(t).wait_send()
            ag_rdma(t).wait_send()

    dma8 = pltpu.SemaphoreType.DMA((N_DEV,))
    return pl.pallas_call(
        body,
        out_shape=jax.ShapeDtypeStruct((B, SQ, DM), jnp.float32),
        in_specs=[
            pl.BlockSpec(memory_space=pltpu.VMEM),
            pl.BlockSpec(memory_space=pltpu.VMEM),
            pl.BlockSpec(memory_space=_ANY),
            pl.BlockSpec(memory_space=_ANY),
            pl.BlockSpec(memory_space=pltpu.VMEM),
        ],
        out_specs=pl.BlockSpec(memory_space=pltpu.VMEM),
        scratch_shapes=[
            pltpu.VMEM((B, HQ_LOC, SKV, DH), jnp.bfloat16),
            pltpu.VMEM((B, HQ_LOC, SKV, DH), jnp.bfloat16),
            pltpu.VMEM((B, HQ_LOC, SQ, DH), jnp.bfloat16),
            pltpu.VMEM((B, SQ, DM), jnp.float32),
            pltpu.VMEM((B, HQ_LOC, 32, 1024), jnp.float32),
            pltpu.VMEM((B, SQ, DM), jnp.bfloat16),
            pltpu.VMEM((N_DEV, B, SQ_RS, DM), jnp.bfloat16),
            pltpu.VMEM((N_DEV, B, SQ_RS, DM), jnp.bfloat16),
            dma8, dma8, dma8, dma8,
            dma8, dma8, dma8, dma8,
            pltpu.SemaphoreType.DMA((2,)),
        ],
        compiler_params=_compiler_params(),
        interpret=(
            pltpu.InterpretParams()
            if os.environ.get("KERNEL_INTERPRET") == "1"
            else False
        ),
    )(xb, wqb, kb, vb, wob)


# device time: 359364 ns/iter; 1.2434x vs baseline; 1.0192x over previous
import os

import jax
import jax.numpy as jnp
from jax import lax
from jax.experimental import pallas as pl
from jax.experimental.pallas import tpu as pltpu

N_DEV = 8
B = 2
SQ = 512
SKV = 4096
SKV_LOC = SKV // N_DEV
HQ_LOC = 8
DH = 64
DM = 768
SQ_RS = SQ // N_DEV

_MESH = pl.DeviceIdType.MESH

_ANY = pl.ANY


def _compiler_params():
    cp = getattr(pltpu, "CompilerParams", None) or getattr(
        pltpu, "TPUCompilerParams"
    )
    return cp(collective_id=0, vmem_limit_bytes=56 * 1024 * 1024)


def kernel(x, Wq, K_ext, V_ext, Wo):
    xb = x.astype(jnp.bfloat16)
    wqb = Wq.astype(jnp.bfloat16)
    kb = (
        K_ext.astype(jnp.bfloat16)
        .reshape(B, SKV_LOC, N_DEV, HQ_LOC, DH)
        .transpose(2, 0, 3, 1, 4)
    )
    vb = (
        V_ext.astype(jnp.bfloat16)
        .reshape(B, SKV_LOC, N_DEV, HQ_LOC, DH)
        .transpose(2, 0, 3, 1, 4)
    )
    wob = Wo.astype(jnp.bfloat16).reshape(HQ_LOC, DH, DM)

    def body(
        x_ref, wq_ref, k_ref, v_ref, wo_ref, out_ref,
        kall, vall, q_ref, part_ref, strip_ref, pbf_ref, rs_ref, ag_ref,
        k_send_sems, k_recv_sems, v_send_sems, v_recv_sems,
        rs_send_sems, rs_recv_sems, ag_send_sems, ag_recv_sems,
        local_sems,
    ):
        me = lax.axis_index("i")

        def peer(off):
            return lax.rem(me + off, N_DEV)

        barrier = pltpu.get_barrier_semaphore()
        for off in range(1, N_DEV):
            pl.semaphore_signal(
                barrier, inc=1, device_id=(peer(off),), device_id_type=_MESH
            )
        pl.semaphore_wait(barrier, N_DEV - 1)

        def a2a_kv(t):
            k_rdma = pltpu.make_async_remote_copy(
                src_ref=k_ref.at[t],
                dst_ref=kall.at[:, :, pl.ds(me * SKV_LOC, SKV_LOC), :],
                send_sem=k_send_sems.at[t],
                recv_sem=k_recv_sems.at[me],
                device_id=(t,),
                device_id_type=_MESH,
            )
            v_rdma = pltpu.make_async_remote_copy(
                src_ref=v_ref.at[t],
                dst_ref=vall.at[:, :, pl.ds(me * SKV_LOC, SKV_LOC), :],
                send_sem=v_send_sems.at[t],
                recv_sem=v_recv_sems.at[me],
                device_id=(t,),
                device_id_type=_MESH,
            )
            return k_rdma, v_rdma

        for off in range(1, N_DEV):
            kr, vr = a2a_kv(peer(off))
            kr.start()
            vr.start()

        kloc = pltpu.make_async_copy(
            k_ref.at[me],
            kall.at[:, :, pl.ds(me * SKV_LOC, SKV_LOC), :],
            local_sems.at[0],
        )
        vloc = pltpu.make_async_copy(
            v_ref.at[me],
            vall.at[:, :, pl.ds(me * SKV_LOC, SKV_LOC), :],
            local_sems.at[1],
        )
        kloc.start()
        vloc.start()

        for b in range(B):
            qb = (
                lax.dot_general(
                    x_ref[b], wq_ref[...],
                    (((1,), (0,)), ((), ())),
                    preferred_element_type=jnp.float32,
                )
                * 0.125
            ).astype(jnp.bfloat16)
            for h in range(HQ_LOC):
                q_ref[b, h] = qb[:, h * DH:(h + 1) * DH]

        kloc.wait()
        vloc.wait()

        def wait_kv(s):
            @pl.when(me != s)
            def _():
                for sems, dst in (
                    (k_recv_sems, kall),
                    (v_recv_sems, vall),
                ):
                    pltpu.make_async_remote_copy(
                        src_ref=k_ref.at[s],
                        dst_ref=dst.at[:, :, pl.ds(s * SKV_LOC, SKV_LOC), :],
                        send_sem=k_send_sems.at[s],
                        recv_sem=sems.at[s],
                        device_id=(s,),
                        device_id_type=_MESH,
                    ).wait_recv()

        KB = 640
        GQ = 32
        qi1 = lax.broadcasted_iota(jnp.int32, (SQ, KB), 0)
        ki1 = lax.broadcasted_iota(jnp.int32, (SQ, KB), 1)
        mask1 = (jnp.abs(qi1 - ki1) <= 128) | (ki1 < 32) | (qi1 < 32)

        part_ref[...] = jnp.zeros((B, SQ, DM), jnp.float32)

        wait_kv(0)
        wait_kv(1)

        def band_step(idx, carry):
            b = idx // HQ_LOC
            h = lax.rem(idx, HQ_LOC)
            q = q_ref[pl.ds(b, 1), pl.ds(h, 1)].reshape((SQ, DH))
            k1 = kall[pl.ds(b, 1), pl.ds(h, 1), pl.ds(0, KB)].reshape((KB, DH))
            v1 = vall[pl.ds(b, 1), pl.ds(h, 1), pl.ds(0, KB)].reshape((KB, DH))
            s1 = lax.dot_general(
                q, k1, (((1,), (1,)), ((), ())),
                preferred_element_type=jnp.float32,
            )
            strip_ref[pl.ds(b, 1), pl.ds(h, 1)] = s1[:GQ][None, None]
            s1 = jnp.where(mask1, s1, -jnp.inf)
            m1 = jnp.max(s1, axis=-1, keepdims=True)
            e1 = jnp.exp(s1 - m1)
            r = 1.0 / jnp.sum(e1, axis=-1, keepdims=True)
            p1 = (e1 * r).astype(jnp.bfloat16)
            c = lax.dot_general(
                p1, v1, (((1,), (0,)), ((), ())),
                preferred_element_type=jnp.float32,
            )
            c = jnp.concatenate(
                [jnp.zeros((GQ, DH), jnp.float32), c[GQ:]]
            ).astype(jnp.bfloat16)
            w = wo_ref[pl.ds(h, 1)].reshape((DH, DM))
            delta = lax.dot_general(
                c, w, (((1,), (0,)), ((), ())),
                preferred_element_type=jnp.float32,
            )
            part_ref[pl.ds(b, 1)] = part_ref[pl.ds(b, 1)] + delta[None]
            return carry

        lax.fori_loop(0, B * HQ_LOC, band_step, 0, unroll=False)

        pbf_ref[...] = part_ref[...].astype(jnp.bfloat16)

        def rs_send(t):
            return pltpu.make_async_remote_copy(
                src_ref=pbf_ref.at[:, pl.ds(t * SQ_RS, SQ_RS), :],
                dst_ref=rs_ref.at[me],
                send_sem=rs_send_sems.at[t],
                recv_sem=rs_recv_sems.at[me],
                device_id=(t,),
                device_id_type=_MESH,
            )

        for off in range(1, N_DEV):
            t = peer(off)

            @pl.when(t != 0)
            def _():
                rs_send(t).start()

        for s in range(2, N_DEV):
            wait_kv(s)

        def strip_step(idx, carry):
            b = idx // HQ_LOC
            h = lax.rem(idx, HQ_LOC)
            q32 = q_ref[pl.ds(b, 1), pl.ds(h, 1), pl.ds(0, GQ)].reshape(
                (GQ, DH)
            )
            k2 = kall[pl.ds(b, 1), pl.ds(h, 1), pl.ds(KB, SKV - KB)].reshape(
                (SKV - KB, DH)
            )
            v2 = vall[pl.ds(b, 1), pl.ds(h, 1), pl.ds(KB, SKV - KB)].reshape(
                (SKV - KB, DH)
            )
            v1 = vall[pl.ds(b, 1), pl.ds(h, 1), pl.ds(0, KB)].reshape((KB, DH))
            s1 = strip_ref[pl.ds(b, 1), pl.ds(h, 1)].reshape((GQ, KB))
            s2 = lax.dot_general(
                q32, k2, (((1,), (1,)), ((), ())),
                preferred_element_type=jnp.float32,
            )
            m = jnp.maximum(
                jnp.max(s1, axis=-1, keepdims=True),
                jnp.max(s2, axis=-1, keepdims=True),
            )
            e1 = jnp.exp(s1 - m)
            e2 = jnp.exp(s2 - m)
            r = 1.0 / (
                jnp.sum(e1, axis=-1, keepdims=True)
                + jnp.sum(e2, axis=-1, keepdims=True)
            )
            p1 = (e1 * r).astype(jnp.bfloat16)
            p2 = (e2 * r).astype(jnp.bfloat16)
            c = (
                lax.dot_general(
                    p1, v1, (((1,), (0,)), ((), ())),
                    preferred_element_type=jnp.float32,
                )
                + lax.dot_general(
                    p2, v2, (((1,), (0,)), ((), ())),
                    preferred_element_type=jnp.float32,
                )
            ).astype(jnp.bfloat16)
            w = wo_ref[pl.ds(h, 1)].reshape((DH, DM))
            delta = lax.dot_general(
                c, w, (((1,), (0,)), ((), ())),
                preferred_element_type=jnp.float32,
            )
            part_ref[pl.ds(b, 1), pl.ds(0, GQ)] = (
                part_ref[pl.ds(b, 1), pl.ds(0, GQ)] + delta[None]
            )
            return carry

        lax.fori_loop(0, B * HQ_LOC, strip_step, 0, unroll=False)

        pbf_ref[:, pl.ds(0, SQ_RS), :] = part_ref[:, pl.ds(0, SQ_RS), :].astype(
            jnp.bfloat16
        )
        for off in range(1, N_DEV):
            t = peer(off)

            @pl.when(t == 0)
            def _():
                rs_send(t).start()

        red = part_ref[:, pl.ds(me * SQ_RS, SQ_RS), :]
        for off in range(1, N_DEV):
            s = peer(off)
            pltpu.make_async_remote_copy(
                src_ref=pbf_ref.at[:, pl.ds(s * SQ_RS, SQ_RS), :],
                dst_ref=rs_ref.at[s],
                send_sem=rs_send_sems.at[s],
                recv_sem=rs_recv_sems.at[s],
                device_id=(s,),
                device_id_type=_MESH,
            ).wait_recv()
            red = red + rs_ref[s].astype(jnp.float32)
        out_ref[:, pl.ds(me * SQ_RS, SQ_RS), :] = red
        ag_ref[pl.ds(me, 1)] = red.astype(jnp.bfloat16)[None]

        def ag_rdma(t):
            return pltpu.make_async_remote_copy(
                src_ref=ag_ref.at[me],
                dst_ref=ag_ref.at[me],
                send_sem=ag_send_sems.at[t],
                recv_sem=ag_recv_sems.at[me],
                device_id=(t,),
                device_id_type=_MESH,
            )

        for off in range(1, N_DEV):
            ag_rdma(peer(off)).start()
        for off in range(1, N_DEV):
            s = peer(off)
            pltpu.make_async_remote_copy(
                src_ref=ag_ref.at[s],
                dst_ref=ag_ref.at[s],
                send_sem=ag_send_sems.at[s],
                recv_sem=ag_recv_sems.at[s],
                device_id=(s,),
                device_id_type=_MESH,
            ).wait_recv()
            out_ref[:, pl.ds(s * SQ_RS, SQ_RS), :] = ag_ref[s].astype(
                jnp.float32
            )

        for off in range(1, N_DEV):
            t = peer(off)
            kr, vr = a2a_kv(t)
            kr.wait_send()
            vr.wait_send()
            rs_send(t).wait_send()
            ag_rdma(t).wait_send()

    dma8 = pltpu.SemaphoreType.DMA((N_DEV,))
    return pl.pallas_call(
        body,
        out_shape=jax.ShapeDtypeStruct((B, SQ, DM), jnp.float32),
        in_specs=[
            pl.BlockSpec(memory_space=pltpu.VMEM),
            pl.BlockSpec(memory_space=pltpu.VMEM),
            pl.BlockSpec(memory_space=_ANY),
            pl.BlockSpec(memory_space=_ANY),
            pl.BlockSpec(memory_space=pltpu.VMEM),
        ],
        out_specs=pl.BlockSpec(memory_space=pltpu.VMEM),
        scratch_shapes=[
            pltpu.VMEM((B, HQ_LOC, SKV, DH), jnp.bfloat16),
            pltpu.VMEM((B, HQ_LOC, SKV, DH), jnp.bfloat16),
            pltpu.VMEM((B, HQ_LOC, SQ, DH), jnp.bfloat16),
            pltpu.VMEM((B, SQ, DM), jnp.float32),
            pltpu.VMEM((B, HQ_LOC, 32, 640), jnp.float32),
            pltpu.VMEM((B, SQ, DM), jnp.bfloat16),
            pltpu.VMEM((N_DEV, B, SQ_RS, DM), jnp.bfloat16),
            pltpu.VMEM((N_DEV, B, SQ_RS, DM), jnp.bfloat16),
            dma8, dma8, dma8, dma8,
            dma8, dma8, dma8, dma8,
            pltpu.SemaphoreType.DMA((2,)),
        ],
        compiler_params=_compiler_params(),
        interpret=(
            pltpu.InterpretParams()
            if os.environ.get("KERNEL_INTERPRET") == "1"
            else False
        ),
    )(xb, wqb, kb, vb, wob)
